# baseline (speedup 1.0000x reference)
import functools

import jax
import jax.numpy as jnp
from jax import lax
from jax.experimental import pallas as pl
from jax.experimental.pallas import tpu as pltpu
from jax.experimental.pallas import tpu_sc as plsc

HIDDEN_DIM = 512
BATCH = 16384
CHUNK = 64


def _make_kernel():
    info = plsc.get_sparse_core_info()
    nw = info.num_cores * info.num_subcores
    b_per_w = BATCH // nw
    n_chunks = b_per_w // CHUNK

    mesh = plsc.VectorSubcoreMesh(core_axis_name="c", subcore_axis_name="s")

    @functools.partial(
        pl.kernel,
        mesh=mesh,
        out_type=jax.ShapeDtypeStruct((BATCH, HIDDEN_DIM), jnp.float32),
        scratch_types=[
            pltpu.VMEM((2, HIDDEN_DIM), jnp.float32),
            pltpu.VMEM((b_per_w + 16,), jnp.int32),
            pltpu.VMEM((CHUNK, HIDDEN_DIM), jnp.float32),
            pltpu.VMEM((CHUNK, HIDDEN_DIM), jnp.float32),
            pltpu.SemaphoreType.DMA,
        ],
    )
    def k(table_hbm, idx_hbm, out_hbm, table_v, idx_v, buf0, buf1, sem_s):
        wid = lax.axis_index("s") * info.num_cores + lax.axis_index("c")
        base = wid * b_per_w
        pltpu.sync_copy(table_hbm, table_v)
        pltpu.sync_copy(idx_hbm.at[pl.ds(base, b_per_w)], idx_v.at[pl.ds(0, b_per_w)])

        bufs = (buf0, buf1)
        stores = [None] * n_chunks
        for c in range(n_chunks):
            if c >= 2:
                stores[c - 2].wait()
            buf = bufs[c % 2]

            def body(r, _, c=c, buf=buf):
                s = idx_v[pl.ds(c * CHUNK + r, 16)][0]
                for v in range(HIDDEN_DIM // 16):
                    sl = pl.ds(v * 16, 16)
                    buf[r, sl] = table_v[s, sl]
                return 0

            lax.fori_loop(0, CHUNK, body, 0)
            stores[c] = pltpu.async_copy(
                buf, out_hbm.at[pl.ds(base + c * CHUNK, CHUNK)], sem_s
            )
        stores[n_chunks - 2].wait()
        stores[n_chunks - 1].wait()

    return k


_lookup = _make_kernel()


def kernel(domain_ids, embed_weight):
    return _lookup(embed_weight, domain_ids.astype(jnp.int32))


# speedup vs baseline: 1.7775x; 1.7775x over previous
import functools

import jax
import jax.numpy as jnp
from jax import lax
from jax.experimental import pallas as pl
from jax.experimental.pallas import tpu as pltpu
from jax.experimental.pallas import tpu_sc as plsc

HIDDEN_DIM = 512
BATCH = 16384
CHUNK = 64
NREP = 16


def _make_kernel():
    info = plsc.get_sparse_core_info()
    nw = info.num_cores * info.num_subcores
    b_per_w = BATCH // nw
    n_chunks = b_per_w // CHUNK

    mesh = plsc.VectorSubcoreMesh(core_axis_name="c", subcore_axis_name="s")

    @functools.partial(
        pl.kernel,
        mesh=mesh,
        out_type=(
            jax.ShapeDtypeStruct((BATCH, HIDDEN_DIM), jnp.float32),
            jax.ShapeDtypeStruct((NREP * nw * 2, HIDDEN_DIM), jnp.float32),
        ),
        scratch_types=[
            pltpu.VMEM((2, HIDDEN_DIM), jnp.float32),
            pltpu.VMEM((b_per_w,), jnp.int32),
            pltpu.VMEM((CHUNK, HIDDEN_DIM), jnp.float32),
            pltpu.VMEM((CHUNK, HIDDEN_DIM), jnp.float32),
            pltpu.VMEM((CHUNK, HIDDEN_DIM), jnp.float32),
            pltpu.SemaphoreType.DMA,
            pltpu.SemaphoreType.DMA,
        ],
    )
    def k(table_hbm, idx_hbm, out_hbm, rep_hbm, table_v, idx_v, buf0, buf1,
          buf2, sem_g, sem_s):
        wid = lax.axis_index("s") * info.num_cores + lax.axis_index("c")
        base = wid * b_per_w
        pltpu.sync_copy(table_hbm, table_v)
        pltpu.sync_copy(idx_hbm.at[pl.ds(base, b_per_w)], idx_v)
        for r in range(NREP):
            pltpu.sync_copy(table_v, rep_hbm.at[pl.ds((r * nw + wid) * 2, 2)])
        pattern = (lax.iota(jnp.int32, 16) % NREP) * (nw * 2) + wid * 2
        for v in range(b_per_w // 16):
            sl = pl.ds(v * 16, 16)
            idx_v[sl] = idx_v[sl] + pattern

        bufs = (buf0, buf1, buf2)
        nbuf = len(bufs)

        def gather(c):
            return pltpu.async_copy(
                rep_hbm.at[idx_v.at[pl.ds(c * CHUNK, CHUNK)]],
                bufs[c % nbuf],
                sem_g,
            )

        gathers = [None] * n_chunks
        stores = [None] * n_chunks
        for c in range(nbuf - 1):
            gathers[c] = gather(c)
        for c in range(n_chunks):
            gathers[c].wait()
            nxt = c + nbuf - 1
            if nxt < n_chunks:
                if nxt >= nbuf:
                    stores[nxt - nbuf].wait()
                gathers[nxt] = gather(nxt)
            stores[c] = pltpu.async_copy(
                bufs[c % nbuf], out_hbm.at[pl.ds(base + c * CHUNK, CHUNK)], sem_s
            )
        for c in range(max(0, n_chunks - nbuf), n_chunks):
            stores[c].wait()

    return k


_lookup = _make_kernel()


def kernel(domain_ids, embed_weight):
    out, _ = _lookup(embed_weight, domain_ids.astype(jnp.int32))
    return out


# speedup vs baseline: 1.7826x; 1.0029x over previous
import functools

import jax
import jax.numpy as jnp
from jax import lax
from jax.experimental import pallas as pl
from jax.experimental.pallas import tpu as pltpu
from jax.experimental.pallas import tpu_sc as plsc

HIDDEN_DIM = 512
BATCH = 16384
CHUNK = 64


def _make_kernel():
    info = plsc.get_sparse_core_info()
    nw = info.num_cores * info.num_subcores
    b_per_w = BATCH // nw
    n_chunks = b_per_w // CHUNK

    mesh = plsc.VectorSubcoreMesh(core_axis_name="c", subcore_axis_name="s")

    @functools.partial(
        pl.kernel,
        mesh=mesh,
        out_type=jax.ShapeDtypeStruct((BATCH, HIDDEN_DIM), jnp.float32),
        scratch_types=[
            pltpu.VMEM((2, HIDDEN_DIM), jnp.float32),
            pltpu.VMEM((b_per_w + 16,), jnp.int32),
            pltpu.VMEM((CHUNK, HIDDEN_DIM), jnp.float32),
            pltpu.VMEM((CHUNK, HIDDEN_DIM), jnp.float32),
            pltpu.VMEM((CHUNK, HIDDEN_DIM), jnp.float32),
            pltpu.SemaphoreType.DMA,
        ],
    )
    def k(table_hbm, idx_hbm, out_hbm, table_v, idx_v, buf0, buf1, buf2, sem_s):
        wid = lax.axis_index("s") * info.num_cores + lax.axis_index("c")
        base = wid * b_per_w
        pltpu.sync_copy(table_hbm, table_v)
        pltpu.sync_copy(
            idx_hbm.at[pl.ds(base, b_per_w)], idx_v.at[pl.ds(0, b_per_w)]
        )

        bufs = (buf0, buf1, buf2)
        nbuf = len(bufs)
        stores = [None] * n_chunks
        for c in range(n_chunks):
            if c >= nbuf:
                stores[c - nbuf].wait()
            buf = bufs[c % nbuf]

            @plsc.parallel_loop(0, CHUNK, 1, unroll=4)
            def body(r, c=c, buf=buf):
                s = idx_v[pl.ds(c * CHUNK + r, 16)][0]
                for v in range(HIDDEN_DIM // 16):
                    sl = pl.ds(v * 16, 16)
                    buf[r, sl] = table_v[s, sl]

            stores[c] = pltpu.async_copy(
                buf, out_hbm.at[pl.ds(base + c * CHUNK, CHUNK)], sem_s
            )
        for c in range(max(0, n_chunks - nbuf), n_chunks):
            stores[c].wait()

    return k


_lookup = _make_kernel()


def kernel(domain_ids, embed_weight):
    return _lookup(embed_weight, domain_ids.astype(jnp.int32))


# speedup vs baseline: 1.9511x; 1.0945x over previous
import functools

import jax
import jax.numpy as jnp
from jax import lax
from jax.experimental import pallas as pl
from jax.experimental.pallas import tpu as pltpu
from jax.experimental.pallas import tpu_sc as plsc

HIDDEN_DIM = 512
BATCH = 16384
CHUNK = 64


def _make_kernel():
    info = plsc.get_sparse_core_info()
    nw = info.num_cores * info.num_subcores
    b_per_w = BATCH // nw
    n_chunks = b_per_w // CHUNK

    mesh = plsc.VectorSubcoreMesh(core_axis_name="c", subcore_axis_name="s")

    @functools.partial(
        pl.kernel,
        mesh=mesh,
        out_type=jax.ShapeDtypeStruct((BATCH, HIDDEN_DIM), jnp.float32),
        scratch_types=[
            pltpu.VMEM((2, HIDDEN_DIM), jnp.float32),
            pltpu.VMEM((b_per_w + 16,), jnp.int32),
            pltpu.VMEM((CHUNK, HIDDEN_DIM), jnp.float32),
            pltpu.VMEM((CHUNK, HIDDEN_DIM), jnp.float32),
            pltpu.VMEM((CHUNK, HIDDEN_DIM), jnp.float32),
            pltpu.SemaphoreType.DMA,
        ],
    )
    def k(table_hbm, idx_hbm, out_hbm, table_v, idx_v, buf0, buf1, buf2, sem_s):
        wid = lax.axis_index("s") * info.num_cores + lax.axis_index("c")
        base = wid * b_per_w
        pltpu.sync_copy(table_hbm, table_v)
        pltpu.sync_copy(
            idx_hbm.at[pl.ds(base, b_per_w)], idx_v.at[pl.ds(0, b_per_w)]
        )

        bufs = (buf0, buf1, buf2)
        nbuf = len(bufs)
        stores = [None] * n_chunks
        for c in range(n_chunks):
            if c >= nbuf:
                stores[c - nbuf].wait()
            buf = bufs[c % nbuf]

            @plsc.parallel_loop(0, CHUNK, 1, unroll=8)
            def body(r, c=c, buf=buf):
                s = idx_v[pl.ds(c * CHUNK + r, 16)][0]
                for v in range(HIDDEN_DIM // 16):
                    sl = pl.ds(v * 16, 16)
                    buf[r, sl] = table_v[s, sl]

            stores[c] = pltpu.async_copy(
                buf, out_hbm.at[pl.ds(base + c * CHUNK, CHUNK)], sem_s
            )
        for c in range(max(0, n_chunks - nbuf), n_chunks):
            stores[c].wait()

    return k


_lookup = _make_kernel()


def kernel(domain_ids, embed_weight):
    return _lookup(embed_weight, domain_ids.astype(jnp.int32))
